# baseline (speedup 1.0000x reference)
import jax
import jax.numpy as jnp
from jax import lax
from jax.experimental import pallas as pl
from jax.experimental.pallas import tpu as pltpu
from jax.experimental.pallas import tpu_sc as plsc

N = 10000
D = 128
E = 320000

NC = 2
NS = 16
NW = NC * NS
R = 320
NPAD = NW * R
C = 4000
NCHUNK = E // C
G = 64
CAP = C + G
NEG = float("-inf")


def _prefix_sum16(x):
    iota = lax.iota(jnp.int32, 16)
    for k in (1, 2, 4, 8):
        idx = jnp.maximum(iota - k, 0)
        sh = x.at[idx].get(mode="promise_in_bounds")
        x = x + jnp.where(iota >= k, sh, 0)
    return x


def _sc_body(hn_hbm, src_hbm, dst_hbm, out_hbm,
             hn_d, acc, srcc, dstc, csrc, cdst, rows, sem):
    wid = lax.axis_index("s") * NC + lax.axis_index("c")
    base = wid * R

    pltpu.sync_copy(hn_hbm.at[pl.ds(base, R)], hn_d.at[pl.ds(0, R)])
    zero16 = jnp.zeros((16,), jnp.float32)
    neg16 = jnp.full((16,), NEG, jnp.float32)
    for f in range(8):
        hn_d[R, pl.ds(f * 16, 16)] = zero16

    def _init_row(i, carry):
        for f in range(8):
            acc[i, pl.ds(f * 16, 16)] = neg16
        return carry

    lax.fori_loop(0, R + 1, _init_row, 0)

    pad_src = jnp.zeros((16,), jnp.int32)
    pad_dst = jnp.full((16,), R, jnp.int32)

    def _chunk(k, carry):
        pltpu.sync_copy(src_hbm.at[pl.ds(k * C, C)], srcc)
        pltpu.sync_copy(dst_hbm.at[pl.ds(k * C, C)], dstc)

        def _compact(i, cursor):
            dv = dstc[pl.ds(i * 16, 16)]
            sv = srcc[pl.ds(i * 16, 16)]
            dl = dv - base
            m = (dl >= 0) & (dl < R)
            cum = _prefix_sum16(jnp.where(m, 1, 0).astype(jnp.int32))
            pos = cursor + cum - 1
            plsc.store_scatter(csrc, [pos], sv, mask=m)
            plsc.store_scatter(cdst, [pos], dl, mask=m)
            return cursor + cum[15]

        count = lax.fori_loop(0, C // 16, _compact, 0)

        for t in range(G // 16):
            csrc[pl.ds(count + t * 16, 16)] = pad_src
            cdst[pl.ds(count + t * 16, 16)] = pad_dst

        ngroups = (count + G - 1) // G

        def _group(g, carry2):
            pltpu.async_copy(
                hn_hbm.at[csrc.at[pl.ds(g * G, G)]], rows, sem).wait()

            def _edges16(q, carry3):
                e0 = q * 16
                dlv = cdst[pl.ds(g * G + e0, 16)]
                for lane in range(16):
                    dloc = dlv[lane]
                    e = e0 + lane
                    for f in range(8):
                        sl = pl.ds(f * 16, 16)
                        hs = rows[e, sl]
                        hd = hn_d[dloc, sl]
                        sg = 1.0 / (1.0 + jnp.exp(-(hs + hd)))
                        acc[dloc, sl] = jnp.maximum(acc[dloc, sl], sg * hs)
                return carry3

            lax.fori_loop(0, G // 16, _edges16, 0)
            return carry2

        lax.fori_loop(0, ngroups, _group, 0)
        return carry

    lax.fori_loop(0, NCHUNK, _chunk, 0)

    pltpu.sync_copy(acc.at[pl.ds(0, R)], out_hbm.at[pl.ds(base, R)])


_sc_segmax = pl.kernel(
    _sc_body,
    out_type=jax.ShapeDtypeStruct((NPAD, D), jnp.float32),
    mesh=plsc.VectorSubcoreMesh(
        core_axis_name="c", subcore_axis_name="s",
        num_cores=NC, num_subcores=NS),
    compiler_params=pltpu.CompilerParams(needs_layout_passes=False),
    scratch_types=[
        pltpu.VMEM((R + 1, D), jnp.float32),
        pltpu.VMEM((R + 1, D), jnp.float32),
        pltpu.VMEM((C,), jnp.int32),
        pltpu.VMEM((C,), jnp.int32),
        pltpu.VMEM((CAP,), jnp.int32),
        pltpu.VMEM((CAP,), jnp.int32),
        pltpu.VMEM((G, D), jnp.float32),
        pltpu.SemaphoreType.DMA,
    ],
)


def _tc_scale_body(h_ref, norm_ref, out_ref):
    out_ref[...] = h_ref[...] * norm_ref[...]


_tc_scale = pl.pallas_call(
    _tc_scale_body,
    grid=(10,),
    in_specs=[pl.BlockSpec((1000, D), lambda i: (i, 0)),
              pl.BlockSpec((1000, 1), lambda i: (i, 0))],
    out_specs=pl.BlockSpec((1000, D), lambda i: (i, 0)),
    out_shape=jax.ShapeDtypeStruct((N, D), jnp.float32),
)


def _tc_finish_body(hn_ref, c_ref, norm_ref, hout_ref, b_ref):
    hn = hn_ref[...]
    c = c_ref[...]
    cf = jnp.where(jnp.isfinite(c), c, 0.0)
    nrm = jnp.sqrt(jnp.sum(hn * hn, axis=1, keepdims=True)
                   + jnp.sum(cf * cf, axis=1, keepdims=True))
    inv = 1.0 / jnp.maximum(nrm, 1e-12)
    hout_ref[...] = cf * norm_ref[...]
    b_ref[...] = jnp.concatenate([hn * inv, cf * inv], axis=1)


_tc_finish = pl.pallas_call(
    _tc_finish_body,
    grid=(10,),
    in_specs=[pl.BlockSpec((1000, D), lambda i: (i, 0)),
              pl.BlockSpec((1000, D), lambda i: (i, 0)),
              pl.BlockSpec((1000, 1), lambda i: (i, 0))],
    out_specs=[pl.BlockSpec((1000, D), lambda i: (i, 0)),
               pl.BlockSpec((1000, 2 * D), lambda i: (i, 0))],
    out_shape=[jax.ShapeDtypeStruct((N, D), jnp.float32),
               jax.ShapeDtypeStruct((N, 2 * D), jnp.float32)],
)


def kernel(h, norm, edge_index):
    h = h.astype(jnp.float32)
    norm = norm.astype(jnp.float32)
    src = edge_index[0].astype(jnp.int32)
    dst = edge_index[1].astype(jnp.int32)
    hn = _tc_scale(h, norm)
    hn_pad = jnp.pad(hn, ((0, NPAD - N), (0, 0)))
    c_raw = _sc_segmax(hn_pad, src, dst)
    h_out, b = _tc_finish(hn, c_raw[:N], norm)
    return (h_out, b)


# speedup vs baseline: 1.3571x; 1.3571x over previous
import jax
import jax.numpy as jnp
from jax import lax
from jax.experimental import pallas as pl
from jax.experimental.pallas import tpu as pltpu
from jax.experimental.pallas import tpu_sc as plsc

N = 10000
D = 128
E = 320000

NC = 2
NS = 16
NW = NC * NS
F = D // NW
W = F * N
CE = 8000
NCHUNK = E // CE
NEG = float("-inf")


def _sc_body(hn_hbm, src_hbm, dst_hbm, out_hbm,
             hn_l, acc, srcb0, srcb1, dstb0, dstb1, sem_e, sem_o):
    srcb = (srcb0, srcb1)
    dstb = (dstb0, dstb1)
    wid = lax.axis_index("s") * NC + lax.axis_index("c")
    base = wid * W

    pltpu.sync_copy(hn_hbm.at[pl.ds(base, W)], hn_l)

    neg16 = jnp.full((16,), NEG, jnp.float32)

    def _init(i, carry):
        for u in range(8):
            acc[pl.ds(i * 128 + u * 16, 16)] = neg16
        return carry

    lax.fori_loop(0, W // 128, _init, 0)

    def _gat(x, idx):
        return x.at[idx].get(mode="promise_in_bounds")

    iota = lax.iota(jnp.int32, 16)
    iota_next = jnp.minimum(iota + 1, 15)
    back = [(k, jnp.maximum(iota - k, 0)) for k in (1, 2, 4, 8)]

    def _groups(slot, nbuf):
        def _group(i, carry):
            sv = srcb[slot][pl.ds(i * 16, 16)]
            dv = dstb[slot][pl.ds(i * 16, 16)]
            kd, vi = plsc.sort_key_val(dv, iota)
            svp = _gat(sv, vi)
            is_end = (kd != _gat(kd, iota_next)) | (iota == 15)
            conds = [(kd == _gat(kd, idxk)) & (iota >= k) for k, idxk in back]
            idxs = [idxk for _, idxk in back]
            for f in range(F):
                sfp = svp + (f * N)
                dfp = kd + (f * N)
                hs = plsc.load_gather(hn_l, [sfp])
                hd = plsc.load_gather(hn_l, [dfp])
                sg = 1.0 / (1.0 + jnp.exp(-(hs + hd)))
                val = sg * hs
                for c, idxk in zip(conds, idxs):
                    val = jnp.where(c, jnp.maximum(val, _gat(val, idxk)), val)
                cur = plsc.load_gather(acc, [dfp])
                plsc.store_scatter(acc, [dfp], jnp.maximum(val, cur),
                                   mask=is_end)
            return carry

        lax.fori_loop(0, CE // 16, _group, nbuf)

    def _start(k, slot):
        pltpu.async_copy(src_hbm.at[pl.ds(k * CE, CE)], srcb[slot], sem_e)
        pltpu.async_copy(dst_hbm.at[pl.ds(k * CE, CE)], dstb[slot], sem_o)

    def _wait(slot):
        pltpu.make_async_copy(src_hbm.at[pl.ds(0, CE)], srcb[slot], sem_e).wait()
        pltpu.make_async_copy(dst_hbm.at[pl.ds(0, CE)], dstb[slot], sem_o).wait()

    _start(0, 0)

    def _pair(j, carry):
        k = j * 2
        _start(k + 1, 1)
        _wait(0)
        _groups(0, carry)
        pl.when(k + 2 < NCHUNK)(lambda: _start(k + 2, 0))
        _wait(1)
        _groups(1, carry)
        return carry

    lax.fori_loop(0, NCHUNK // 2, _pair, 0)

    pltpu.sync_copy(acc, out_hbm.at[pl.ds(base, W)])


_sc_segmax = pl.kernel(
    _sc_body,
    out_type=jax.ShapeDtypeStruct((N * D,), jnp.float32),
    mesh=plsc.VectorSubcoreMesh(
        core_axis_name="c", subcore_axis_name="s",
        num_cores=NC, num_subcores=NS),
    compiler_params=pltpu.CompilerParams(needs_layout_passes=False),
    scratch_types=[
        pltpu.VMEM((W,), jnp.float32),
        pltpu.VMEM((W,), jnp.float32),
        pltpu.VMEM((CE,), jnp.int32),
        pltpu.VMEM((CE,), jnp.int32),
        pltpu.VMEM((CE,), jnp.int32),
        pltpu.VMEM((CE,), jnp.int32),
        pltpu.SemaphoreType.DMA,
        pltpu.SemaphoreType.DMA,
    ],
)


def _tc_scale_body(h_ref, norm_ref, out_ref):
    out_ref[...] = h_ref[...] * norm_ref[...]


_tc_scale = pl.pallas_call(
    _tc_scale_body,
    grid=(10,),
    in_specs=[pl.BlockSpec((1000, D), lambda i: (i, 0)),
              pl.BlockSpec((1000, 1), lambda i: (i, 0))],
    out_specs=pl.BlockSpec((1000, D), lambda i: (i, 0)),
    out_shape=jax.ShapeDtypeStruct((N, D), jnp.float32),
)


def _tc_finish_body(hn_ref, c_ref, norm_ref, hout_ref, b_ref):
    hn = hn_ref[...]
    c = c_ref[...]
    cf = jnp.where(jnp.isfinite(c), c, 0.0)
    nrm = jnp.sqrt(jnp.sum(hn * hn, axis=1, keepdims=True)
                   + jnp.sum(cf * cf, axis=1, keepdims=True))
    inv = 1.0 / jnp.maximum(nrm, 1e-12)
    hout_ref[...] = cf * norm_ref[...]
    b_ref[...] = jnp.concatenate([hn * inv, cf * inv], axis=1)


_tc_finish = pl.pallas_call(
    _tc_finish_body,
    grid=(10,),
    in_specs=[pl.BlockSpec((1000, D), lambda i: (i, 0)),
              pl.BlockSpec((1000, D), lambda i: (i, 0)),
              pl.BlockSpec((1000, 1), lambda i: (i, 0))],
    out_specs=[pl.BlockSpec((1000, D), lambda i: (i, 0)),
               pl.BlockSpec((1000, 2 * D), lambda i: (i, 0))],
    out_shape=[jax.ShapeDtypeStruct((N, D), jnp.float32),
               jax.ShapeDtypeStruct((N, 2 * D), jnp.float32)],
)


def kernel(h, norm, edge_index):
    h = h.astype(jnp.float32)
    norm = norm.astype(jnp.float32)
    src = edge_index[0].astype(jnp.int32)
    dst = edge_index[1].astype(jnp.int32)
    hn = _tc_scale(h, norm)
    hn_t = jnp.transpose(hn).reshape(-1)
    c_t = _sc_segmax(hn_t, src, dst)
    c = jnp.transpose(c_t.reshape(D, N))
    h_out, b = _tc_finish(hn, c, norm)
    return (h_out, b)


# speedup vs baseline: 3.1882x; 2.3493x over previous
import jax
import jax.numpy as jnp
from jax import lax
from jax.experimental import pallas as pl
from jax.experimental.pallas import tpu as pltpu
from jax.experimental.pallas import tpu_sc as plsc

N = 10000
D = 128
E = 320000

NC = 2
NS = 16
NW = NC * NS
F = D // NW
W = F * N
CE = 8000
NCHUNK = E // CE
B = 4
NEG = float("-inf")


def _sc_body(hn_hbm, src_hbm, dst_hbm, out_hbm,
             hn_l, acc, srcb0, srcb1, dstb0, dstb1, sem_e, sem_o):
    srcb = (srcb0, srcb1)
    dstb = (dstb0, dstb1)
    wid = lax.axis_index("s") * NC + lax.axis_index("c")
    base = wid * W

    pltpu.sync_copy(hn_hbm.at[pl.ds(base, W)], hn_l)

    neg16 = jnp.full((16,), NEG, jnp.float32)

    def _init(i, carry):
        for u in range(8):
            acc[pl.ds(i * 128 + u * 16, 16)] = neg16
        return carry

    lax.fori_loop(0, W // 128, _init, 0)

    def _gat(x, idx):
        return x.at[idx].get(mode="promise_in_bounds")

    iota = lax.iota(jnp.int32, 16)
    iota_next = jnp.minimum(iota + 1, 15)
    back = [(k, jnp.maximum(iota - k, 0)) for k in (1, 2, 4, 8)]

    def _groups(slot, nbuf):
        def _batch(ib, carry):
            ge = []
            for g in range(B):
                sv = srcb[slot][pl.ds(ib * (16 * B) + g * 16, 16)]
                dv = dstb[slot][pl.ds(ib * (16 * B) + g * 16, 16)]
                kd, vi = plsc.sort_key_val(dv, iota)
                svp = _gat(sv, vi)
                is_end = (kd != _gat(kd, iota_next)) | (iota == 15)
                conds = [(kd == _gat(kd, idxk)) & (iota >= k)
                         for k, idxk in back]
                dfps, vals = [], []
                for f in range(F):
                    sfp = svp + (f * N)
                    dfp = kd + (f * N)
                    hs = plsc.load_gather(hn_l, [sfp])
                    hd = plsc.load_gather(hn_l, [dfp])
                    sg = 1.0 / (1.0 + jnp.exp(-(hs + hd)))
                    val = sg * hs
                    for c, (_, idxk) in zip(conds, back):
                        val = jnp.where(
                            c, jnp.maximum(val, _gat(val, idxk)), val)
                    cur = plsc.load_gather(acc, [dfp])
                    dfps.append(dfp)
                    vals.append(jnp.maximum(val, cur))
                ge.append((is_end, dfps, vals))
            for is_end, dfps, vals in ge:
                for f in range(F):
                    plsc.store_scatter(acc, [dfps[f]], vals[f], mask=is_end)
            pend = []
            for is_end, dfps, vals in ge:
                for f in range(F):
                    chk = plsc.load_gather(acc, [dfps[f]])
                    pend.append(is_end & (chk < vals[f]))

            def _any(ps):
                a = ps[0]
                for p in ps[1:]:
                    a = a | p
                return jnp.any(a)

            def _step(ps):
                j = 0
                for is_end, dfps, vals in ge:
                    for f in range(F):
                        plsc.store_scatter(acc, [dfps[f]], vals[f],
                                           mask=ps[j])
                        j += 1
                out = []
                j = 0
                for is_end, dfps, vals in ge:
                    for f in range(F):
                        chk = plsc.load_gather(acc, [dfps[f]])
                        out.append(is_end & (chk < vals[f]))
                        j += 1
                return tuple(out)

            lax.while_loop(_any, _step, tuple(pend))
            return carry

        lax.fori_loop(0, CE // (16 * B), _batch, nbuf)

    def _start(k, slot):
        pltpu.async_copy(src_hbm.at[pl.ds(k * CE, CE)], srcb[slot], sem_e)
        pltpu.async_copy(dst_hbm.at[pl.ds(k * CE, CE)], dstb[slot], sem_o)

    def _wait(slot):
        pltpu.make_async_copy(src_hbm.at[pl.ds(0, CE)], srcb[slot], sem_e).wait()
        pltpu.make_async_copy(dst_hbm.at[pl.ds(0, CE)], dstb[slot], sem_o).wait()

    _start(0, 0)

    def _pair(j, carry):
        k = j * 2
        _start(k + 1, 1)
        _wait(0)
        _groups(0, carry)
        pl.when(k + 2 < NCHUNK)(lambda: _start(k + 2, 0))
        _wait(1)
        _groups(1, carry)
        return carry

    lax.fori_loop(0, NCHUNK // 2, _pair, 0)

    pltpu.sync_copy(acc, out_hbm.at[pl.ds(base, W)])


_sc_segmax = pl.kernel(
    _sc_body,
    out_type=jax.ShapeDtypeStruct((N * D,), jnp.float32),
    mesh=plsc.VectorSubcoreMesh(
        core_axis_name="c", subcore_axis_name="s",
        num_cores=NC, num_subcores=NS),
    compiler_params=pltpu.CompilerParams(needs_layout_passes=False),
    scratch_types=[
        pltpu.VMEM((W,), jnp.float32),
        pltpu.VMEM((W,), jnp.float32),
        pltpu.VMEM((CE,), jnp.int32),
        pltpu.VMEM((CE,), jnp.int32),
        pltpu.VMEM((CE,), jnp.int32),
        pltpu.VMEM((CE,), jnp.int32),
        pltpu.SemaphoreType.DMA,
        pltpu.SemaphoreType.DMA,
    ],
)


def _tc_scale_body(h_ref, norm_ref, out_ref):
    out_ref[...] = h_ref[...] * norm_ref[...]


_tc_scale = pl.pallas_call(
    _tc_scale_body,
    grid=(10,),
    in_specs=[pl.BlockSpec((1000, D), lambda i: (i, 0)),
              pl.BlockSpec((1000, 1), lambda i: (i, 0))],
    out_specs=pl.BlockSpec((1000, D), lambda i: (i, 0)),
    out_shape=jax.ShapeDtypeStruct((N, D), jnp.float32),
)


def _tc_finish_body(hn_ref, c_ref, norm_ref, hout_ref, b_ref):
    hn = hn_ref[...]
    c = c_ref[...]
    cf = jnp.where(jnp.isfinite(c), c, 0.0)
    nrm = jnp.sqrt(jnp.sum(hn * hn, axis=1, keepdims=True)
                   + jnp.sum(cf * cf, axis=1, keepdims=True))
    inv = 1.0 / jnp.maximum(nrm, 1e-12)
    hout_ref[...] = cf * norm_ref[...]
    b_ref[...] = jnp.concatenate([hn * inv, cf * inv], axis=1)


_tc_finish = pl.pallas_call(
    _tc_finish_body,
    grid=(10,),
    in_specs=[pl.BlockSpec((1000, D), lambda i: (i, 0)),
              pl.BlockSpec((1000, D), lambda i: (i, 0)),
              pl.BlockSpec((1000, 1), lambda i: (i, 0))],
    out_specs=[pl.BlockSpec((1000, D), lambda i: (i, 0)),
               pl.BlockSpec((1000, 2 * D), lambda i: (i, 0))],
    out_shape=[jax.ShapeDtypeStruct((N, D), jnp.float32),
               jax.ShapeDtypeStruct((N, 2 * D), jnp.float32)],
)


def kernel(h, norm, edge_index):
    h = h.astype(jnp.float32)
    norm = norm.astype(jnp.float32)
    src = edge_index[0].astype(jnp.int32)
    dst = edge_index[1].astype(jnp.int32)
    hn = _tc_scale(h, norm)
    hn_t = jnp.transpose(hn).reshape(-1)
    c_t = _sc_segmax(hn_t, src, dst)
    c = jnp.transpose(c_t.reshape(D, N))
    h_out, b = _tc_finish(hn, c, norm)
    return (h_out, b)
